# baseline (speedup 1.0000x reference)
import jax
import jax.numpy as jnp
from jax.experimental import pallas as pl
from jax.experimental.pallas import tpu as pltpu

_RADIUS = 2
_TOP_K = 500
_KER = 4
_H = 1024
_W = 1024
_NTH = _H // _KER
_NTW = _W // _KER
_C = 64
_PAD_K = 512


def _topk_kernel(s_ref, vals_ref, idx_ref, vscr, pscr, rmax):
    s = s_ref[:, :]
    row = jax.lax.broadcasted_iota(jnp.int32, (_H, _W), 0)
    col = jax.lax.broadcasted_iota(jnp.int32, (_H, _W), 1)
    r = _RADIUS + 1
    inb = (row >= r) & (row < _H - r) & (col >= r) & (col < _W - r)
    s = jnp.where(inb, s, 0.0)

    t_l = jax.lax.broadcasted_iota(jnp.int32, (_NTH, _H), 0)
    j_l = jax.lax.broadcasted_iota(jnp.int32, (_NTH, _H), 1)
    j_r = jax.lax.broadcasted_iota(jnp.int32, (_W, _NTW), 0)
    t_r = jax.lax.broadcasted_iota(jnp.int32, (_W, _NTW), 1)
    dn = (((1,), (0,)), ((), ()))
    rows = []
    for k1 in range(_KER):
        sel_l = (j_l == _KER * t_l + k1).astype(jnp.float32)
        rows.append(
            jax.lax.dot_general(
                sel_l,
                s,
                dn,
                precision=jax.lax.Precision.HIGHEST,
                preferred_element_type=jnp.float32,
            )
        )
    val = None
    idx = jnp.zeros((_NTH, _NTW), jnp.int32)
    for k1 in range(_KER):
        for k2 in range(_KER):
            sel_r = (j_r == _KER * t_r + k2).astype(jnp.float32)
            cand = jax.lax.dot_general(
                rows[k1],
                sel_r,
                dn,
                precision=jax.lax.Precision.HIGHEST,
                preferred_element_type=jnp.float32,
            )
            k = k1 * _KER + k2
            if k == 0:
                val = cand
            else:
                take = cand > val
                val = jnp.where(take, cand, val)
                idx = jnp.where(take, k, idx)

    tr = jax.lax.broadcasted_iota(jnp.int32, (_NTH, _NTW), 0)
    tc = jax.lax.broadcasted_iota(jnp.int32, (_NTH, _NTW), 1)
    prow = tr * _KER + idx // _KER
    pcol = tc * _KER + idx % _KER
    pidx = prow * _W + pcol

    vscr[:, :] = val
    pscr[:, :] = pidx
    rmax[:, :] = jnp.max(val, axis=1, keepdims=True)

    lane = jax.lax.broadcasted_iota(jnp.int32, (1, _NTW), 1)
    subl = jax.lax.broadcasted_iota(jnp.int32, (_NTH, 1), 0)

    def body(i, carry):
        rm = rmax[:, :]
        gm = jnp.max(rm)
        rr = jnp.min(jnp.where(rm == gm, subl, _NTH))
        vrow = vscr[pl.ds(rr, 1), :]
        cc = jnp.min(jnp.where(vrow == gm, lane, _NTW))
        prow_ = pscr[pl.ds(rr, 1), :]
        p = jnp.sum(jnp.where(lane == cc, prow_, 0))
        vals_ref[pl.ds(i, 1), :] = jnp.reshape(gm, (1, 1))
        idx_ref[pl.ds(i, 1), :] = jnp.reshape(p, (1, 1))
        nrow = jnp.where(lane == cc, -1.0, vrow)
        vscr[pl.ds(rr, 1), :] = nrow
        rmax[pl.ds(rr, 1), :] = jnp.reshape(jnp.max(nrow), (1, 1))
        return carry

    jax.lax.fori_loop(0, _TOP_K, body, 0)


def _gather_kernel(rows_ref, cols_ref, d_ref, out_ref):
    n = pl.program_id(0)
    rsub = rows_ref[n] % 8
    csub = cols_ref[n] % 128
    d = d_ref[:, :, :]
    si = jax.lax.broadcasted_iota(jnp.int32, (_C, 8, 128), 1)
    li = jax.lax.broadcasted_iota(jnp.int32, (_C, 8, 128), 2)
    hit = (si == rsub) & (li == csub)
    v = jnp.sum(jnp.where(hit, d, 0.0), axis=(1, 2))[:, None]
    nrm = jnp.sqrt(jnp.sum(v * v))
    out_ref[:, :, :] = jnp.reshape(v / nrm, (1, _C, 1))


def _impl(scores_map, descriptor_map):
    scores = scores_map.reshape(_H, _W)
    vals, idx = pl.pallas_call(
        _topk_kernel,
        out_shape=(
            jax.ShapeDtypeStruct((_PAD_K, 1), jnp.float32),
            jax.ShapeDtypeStruct((_PAD_K, 1), jnp.int32),
        ),
        scratch_shapes=[
            pltpu.VMEM((_NTH, _NTW), jnp.float32),
            pltpu.VMEM((_NTH, _NTW), jnp.int32),
            pltpu.VMEM((_NTH, 1), jnp.float32),
        ],
    )(scores)
    vals = vals[:_TOP_K, 0]
    idx = idx[:_TOP_K, 0]
    rowsk = idx // _W
    colsk = idx % _W
    keypoints_xy = jnp.stack([colsk, rowsk], axis=1)

    d = descriptor_map.reshape(_C, _H, _W)
    desc = pl.pallas_call(
        _gather_kernel,
        grid_spec=pltpu.PrefetchScalarGridSpec(
            num_scalar_prefetch=2,
            grid=(_TOP_K,),
            in_specs=[
                pl.BlockSpec(
                    (_C, 8, 128),
                    lambda n, rows, cols: (0, rows[n] // 8, cols[n] // 128),
                ),
            ],
            out_specs=pl.BlockSpec(
                (1, _C, 1), lambda n, rows, cols: (n, 0, 0)
            ),
        ),
        out_shape=jax.ShapeDtypeStruct((_TOP_K, _C, 1), jnp.float32),
    )(rowsk, colsk, d)
    return keypoints_xy, desc[:, :, 0], vals


_impl_jit = jax.jit(_impl)


def kernel(scores_map, descriptor_map):
    return _impl_jit(scores_map, descriptor_map)


# speedup vs baseline: 1.0334x; 1.0334x over previous
import jax
import jax.numpy as jnp
from jax.experimental import pallas as pl
from jax.experimental.pallas import tpu as pltpu

_RADIUS = 2
_TOP_K = 500
_KER = 4
_H = 1024
_W = 1024
_NTH = _H // _KER
_NTW = _W // _KER
_C = 64
_PAD_K = 512


def _topk_kernel(s_ref, vals_ref, idx_ref, vscr, pscr):
    s = s_ref[:, :]
    row = jax.lax.broadcasted_iota(jnp.int32, (_H, _W), 0)
    col = jax.lax.broadcasted_iota(jnp.int32, (_H, _W), 1)
    r = _RADIUS + 1
    inb = (row >= r) & (row < _H - r) & (col >= r) & (col < _W - r)
    s = jnp.where(inb, s, 0.0)

    s3 = jnp.reshape(s, (_NTH, _KER, _W))
    rmax4 = jnp.max(s3, axis=1)
    z1 = jnp.maximum(rmax4, pltpu.roll(rmax4, _W - 1, axis=1))
    z2 = jnp.maximum(z1, pltpu.roll(z1, _W - 2, axis=1))
    lane1024r = jax.lax.broadcasted_iota(jnp.int32, (_NTH, _W), 1)
    is4 = lane1024r % _KER == 0
    zm = jnp.where(is4, z2, -1.0)
    b1 = jnp.maximum(zm, pltpu.roll(zm, 1, axis=1))
    tmax = jnp.maximum(b1, pltpu.roll(b1, 2, axis=1))

    k1i = jax.lax.broadcasted_iota(jnp.int32, (_NTH, _KER, _W), 1)
    rank = k1i * _KER + (lane1024r % _KER)[:, None, :]
    eq = s3 == tmax[:, None, :]
    cand_rank = jnp.where(eq, rank, 64)
    rmin_rows = jnp.min(cand_rank, axis=1)
    r1 = jnp.minimum(rmin_rows, pltpu.roll(rmin_rows, _W - 1, axis=1))
    wrank = jnp.minimum(r1, pltpu.roll(r1, _W - 2, axis=1))

    tr1024 = jax.lax.broadcasted_iota(jnp.int32, (_NTH, _W), 0)
    pidx = (_KER * tr1024 + wrank // _KER) * _W + lane1024r + wrank % _KER
    vscr[:, :] = jnp.where(is4, z2, -1.0)
    pscr[:, :] = jnp.where(is4, pidx, 0)

    lane1024 = jax.lax.broadcasted_iota(jnp.int32, (1, _W), 1)
    lane256 = jax.lax.broadcasted_iota(jnp.int32, (1, _NTH), 1)

    rmax0 = jnp.max(vscr[:, :], axis=1)[None, :]

    def body(i, rmaxv):
        gm = jnp.max(rmaxv)
        rr = jnp.min(jnp.where(rmaxv == gm, lane256, _NTH))
        vrow = vscr[pl.ds(rr, 1), :]
        cc = jnp.min(jnp.where(vrow == gm, lane1024, _W))
        prow_ = pscr[pl.ds(rr, 1), :]
        p = jnp.sum(jnp.where(lane1024 == cc, prow_, 0))
        vals_ref[pl.ds(i, 1), :] = jnp.reshape(gm, (1, 1))
        idx_ref[pl.ds(i, 1), :] = jnp.reshape(p, (1, 1))
        nrow = jnp.where(lane1024 == cc, -1.0, vrow)
        vscr[pl.ds(rr, 1), :] = nrow
        return jnp.where(lane256 == rr, jnp.max(nrow), rmaxv)

    jax.lax.fori_loop(0, _TOP_K, body, rmax0)


def _gather_kernel(rows_ref, cols_ref, d_ref, out_ref):
    n = pl.program_id(0)
    rsub = rows_ref[n] % 8
    csub = cols_ref[n] % 128
    d = d_ref[:, :, :]
    si = jax.lax.broadcasted_iota(jnp.int32, (_C, 8, 128), 1)
    li = jax.lax.broadcasted_iota(jnp.int32, (_C, 8, 128), 2)
    hit = (si == rsub) & (li == csub)
    v = jnp.sum(jnp.where(hit, d, 0.0), axis=(1, 2))[:, None]
    nrm = jnp.sqrt(jnp.sum(v * v))
    out_ref[:, :, :] = jnp.reshape(v / nrm, (1, _C, 1))


def _impl(scores_map, descriptor_map):
    scores = scores_map.reshape(_H, _W)
    vals, idx = pl.pallas_call(
        _topk_kernel,
        out_shape=(
            jax.ShapeDtypeStruct((_PAD_K, 1), jnp.float32),
            jax.ShapeDtypeStruct((_PAD_K, 1), jnp.int32),
        ),
        scratch_shapes=[
            pltpu.VMEM((_NTH, _W), jnp.float32),
            pltpu.VMEM((_NTH, _W), jnp.int32),
        ],
    )(scores)
    vals = vals[:_TOP_K, 0]
    idx = idx[:_TOP_K, 0]
    rowsk = idx // _W
    colsk = idx % _W
    keypoints_xy = jnp.stack([colsk, rowsk], axis=1)

    d = descriptor_map.reshape(_C, _H, _W)
    desc = pl.pallas_call(
        _gather_kernel,
        grid_spec=pltpu.PrefetchScalarGridSpec(
            num_scalar_prefetch=2,
            grid=(_TOP_K,),
            in_specs=[
                pl.BlockSpec(
                    (_C, 8, 128),
                    lambda n, rows, cols: (0, rows[n] // 8, cols[n] // 128),
                ),
            ],
            out_specs=pl.BlockSpec(
                (1, _C, 1), lambda n, rows, cols: (n, 0, 0)
            ),
        ),
        out_shape=jax.ShapeDtypeStruct((_TOP_K, _C, 1), jnp.float32),
    )(rowsk, colsk, d)
    return keypoints_xy, desc[:, :, 0], vals


_impl_jit = jax.jit(_impl)


def kernel(scores_map, descriptor_map):
    return _impl_jit(scores_map, descriptor_map)


# speedup vs baseline: 1.4755x; 1.4278x over previous
import jax
import jax.numpy as jnp
from jax.experimental import pallas as pl
from jax.experimental.pallas import tpu as pltpu

_RADIUS = 2
_TOP_K = 500
_KER = 4
_H = 1024
_W = 1024
_NTH = _H // _KER
_NTW = _W // _KER
_C = 64
_PAD_K = 512


def _topk_kernel(s_ref, vals_ref, idx_ref, vscr, pscr):
    s = s_ref[:, :]
    row = jax.lax.broadcasted_iota(jnp.int32, (_H, _W), 0)
    col = jax.lax.broadcasted_iota(jnp.int32, (_H, _W), 1)
    r = _RADIUS + 1
    inb = (row >= r) & (row < _H - r) & (col >= r) & (col < _W - r)
    s = jnp.where(inb, s, 0.0)

    s3 = jnp.reshape(s, (_NTH, _KER, _W))
    rmax4 = jnp.max(s3, axis=1)
    z1 = jnp.maximum(rmax4, pltpu.roll(rmax4, _W - 1, axis=1))
    z2 = jnp.maximum(z1, pltpu.roll(z1, _W - 2, axis=1))
    lane1024r = jax.lax.broadcasted_iota(jnp.int32, (_NTH, _W), 1)
    is4 = lane1024r % _KER == 0
    zm = jnp.where(is4, z2, -1.0)
    b1 = jnp.maximum(zm, pltpu.roll(zm, 1, axis=1))
    tmax = jnp.maximum(b1, pltpu.roll(b1, 2, axis=1))

    k1i = jax.lax.broadcasted_iota(jnp.int32, (_NTH, _KER, _W), 1)
    rank = k1i * _KER + (lane1024r % _KER)[:, None, :]
    eq = s3 == tmax[:, None, :]
    cand_rank = jnp.where(eq, rank, 64)
    rmin_rows = jnp.min(cand_rank, axis=1)
    r1 = jnp.minimum(rmin_rows, pltpu.roll(rmin_rows, _W - 1, axis=1))
    wrank = jnp.minimum(r1, pltpu.roll(r1, _W - 2, axis=1))

    tr1024 = jax.lax.broadcasted_iota(jnp.int32, (_NTH, _W), 0)
    pidx = (_KER * tr1024 + wrank // _KER) * _W + lane1024r + wrank % _KER
    vscr[:, :] = jnp.where(is4, z2, -1.0)
    pscr[:, :] = jnp.where(is4, pidx, 0)

    lane1024 = jax.lax.broadcasted_iota(jnp.int32, (1, _W), 1)
    lane256 = jax.lax.broadcasted_iota(jnp.int32, (1, _NTH), 1)

    rmax0 = jnp.max(vscr[:, :], axis=1)[None, :]

    def body(i, rmaxv):
        gm = jnp.max(rmaxv)
        rr = jnp.min(jnp.where(rmaxv == gm, lane256, _NTH))
        vrow = vscr[pl.ds(rr, 1), :]
        cc = jnp.min(jnp.where(vrow == gm, lane1024, _W))
        prow_ = pscr[pl.ds(rr, 1), :]
        p = jnp.sum(jnp.where(lane1024 == cc, prow_, 0))
        vals_ref[pl.ds(i, 1), :] = jnp.reshape(gm, (1, 1))
        idx_ref[pl.ds(i, 1), :] = jnp.reshape(p, (1, 1))
        nrow = jnp.where(lane1024 == cc, -1.0, vrow)
        vscr[pl.ds(rr, 1), :] = nrow
        return jnp.where(lane256 == rr, jnp.max(nrow), rmaxv)

    jax.lax.fori_loop(0, _TOP_K, body, rmax0)


_GB = 4


def _gather_kernel(rows_ref, cols_ref, d0, d1, d2, d3, out_ref):
    n = pl.program_id(0)
    si = jax.lax.broadcasted_iota(jnp.int32, (_C, 8, 128), 1)
    li = jax.lax.broadcasted_iota(jnp.int32, (_C, 8, 128), 2)
    outs = []
    for j, dref in enumerate((d0, d1, d2, d3)):
        rsub = rows_ref[_GB * n + j] % 8
        csub = cols_ref[_GB * n + j] % 128
        hit = (si == rsub) & (li == csub)
        v = jnp.sum(jnp.where(hit, dref[:, :, :], 0.0), axis=(1, 2))
        nrm = jnp.sqrt(jnp.sum(v * v))
        outs.append(jnp.reshape(v / nrm, (1, _C, 1)))
    out_ref[:, :, :] = jnp.concatenate(outs, axis=0)


def _impl(scores_map, descriptor_map):
    scores = scores_map.reshape(_H, _W)
    vals, idx = pl.pallas_call(
        _topk_kernel,
        out_shape=(
            jax.ShapeDtypeStruct((_PAD_K, 1), jnp.float32),
            jax.ShapeDtypeStruct((_PAD_K, 1), jnp.int32),
        ),
        scratch_shapes=[
            pltpu.VMEM((_NTH, _W), jnp.float32),
            pltpu.VMEM((_NTH, _W), jnp.int32),
        ],
    )(scores)
    vals = vals[:_TOP_K, 0]
    idx = idx[:_TOP_K, 0]
    rowsk = idx // _W
    colsk = idx % _W
    keypoints_xy = jnp.stack([colsk, rowsk], axis=1)

    d = descriptor_map.reshape(_C, _H, _W)
    desc = pl.pallas_call(
        _gather_kernel,
        grid_spec=pltpu.PrefetchScalarGridSpec(
            num_scalar_prefetch=2,
            grid=(_TOP_K // _GB,),
            in_specs=[
                pl.BlockSpec(
                    (_C, 8, 128),
                    lambda n, rows, cols, j=j: (
                        0,
                        rows[_GB * n + j] // 8,
                        cols[_GB * n + j] // 128,
                    ),
                )
                for j in range(_GB)
            ],
            out_specs=pl.BlockSpec(
                (_GB, _C, 1), lambda n, rows, cols: (n, 0, 0)
            ),
        ),
        out_shape=jax.ShapeDtypeStruct((_TOP_K, _C, 1), jnp.float32),
    )(rowsk, colsk, d, d, d, d)
    return keypoints_xy, desc[:, :, 0], vals


_impl_jit = jax.jit(_impl)


def kernel(scores_map, descriptor_map):
    return _impl_jit(scores_map, descriptor_map)
